# baseline (speedup 1.0000x reference)
import functools

import jax
import jax.numpy as jnp
from jax import lax
from jax.experimental import pallas as pl
from jax.experimental.pallas import tpu as pltpu
from jax.experimental.pallas import tpu_sc as plsc

E = 8
K = 2
T = 2048
D = 1024
F = 1408
BS = 128

P = T * K
BM = 640
G = -(-P // BM) + E - 1
N = G * BM
NF = F // BS


def _sc_row_gather(src, idx, n_rows, chunk, width, dtype):
    info = plsc.get_sparse_core_info()
    nw = info.num_cores * info.num_subcores
    b_per_w = n_rows // nw
    n_chunks = b_per_w // chunk
    mesh = plsc.VectorSubcoreMesh(core_axis_name="c", subcore_axis_name="s")

    @functools.partial(
        pl.kernel,
        mesh=mesh,
        out_type=jax.ShapeDtypeStruct((n_rows, width), dtype),
        scratch_types=[
            pltpu.VMEM((b_per_w,), jnp.int32),
            pltpu.VMEM((chunk, width), dtype),
            pltpu.SemaphoreType.DMA,
        ],
    )
    def gather_k(src_hbm, idx_hbm, out_hbm, idx_v, buf, sem):
        wid = lax.axis_index("s") * info.num_cores + lax.axis_index("c")
        base = wid * b_per_w
        pltpu.sync_copy(idx_hbm.at[pl.ds(base, b_per_w)], idx_v)

        def body(c, _):
            off = c * chunk
            pltpu.async_copy(
                src_hbm.at[idx_v.at[pl.ds(off, chunk)]], buf, sem
            ).wait()
            pltpu.sync_copy(buf, out_hbm.at[pl.ds(base + off, chunk)])
            return 0

        lax.fori_loop(0, n_chunks, body, 0)

    return gather_k(src, idx)


def _mlp_body(ebk_ref, nb_ref, xs_ref, w0_ref, w1_ref, w2_ref,
              s0_ref, s1_ref, s2_ref, out_ref, xsb_ref, wgu_ref,
              hi_ref, wd_ref):
    g = pl.program_id(0)
    fb = pl.program_id(1)

    @pl.when(jnp.logical_and(g < nb_ref[0], fb == 0))
    def _():
        xsb_ref[...] = xs_ref[...].astype(jnp.bfloat16)

    dn = (((1,), (1,)), ((), ()))
    bf = jnp.bfloat16

    @pl.when(jnp.logical_and(g < nb_ref[0], fb < NF))
    def _():
        s0_full = s0_ref[0, 0]
        s1_full = s1_ref[0, 0]
        expand = (lax.broadcasted_iota(jnp.int32, (D, D // BS), 0) // BS
                  == lax.broadcasted_iota(jnp.int32, (D, D // BS), 1)
                  ).astype(jnp.float32)
        s2_full = lax.dot_general(expand, s2_ref[0, 0], dn,
                                  preferred_element_type=jnp.float32)
        slot = lax.rem(fb, 2)
        wgu_ref[slot, 0:BS, :] = (w0_ref[0] * s0_full).astype(bf)
        wgu_ref[slot, BS:2 * BS, :] = (w1_ref[0] * s1_full).astype(bf)
        off = pl.multiple_of(fb * BS, BS)
        wd_ref[:, pl.ds(off, BS)] = (w2_ref[0] * s2_full).astype(bf)

    @pl.when(jnp.logical_and(g < nb_ref[0], fb >= 1))
    def _():
        xs = xsb_ref[...]
        ft = fb - 1
        slot = lax.rem(ft, 2)
        gu = lax.dot_general(xs, wgu_ref[slot], dn,
                             preferred_element_type=jnp.float32)
        gi = gu[:, :BS]
        ui = gu[:, BS:]
        hi = (gi * jax.nn.sigmoid(gi) * ui).astype(bf)
        off = pl.multiple_of(ft * BS, BS)
        hi_ref[:, pl.ds(off, BS)] = hi

    @pl.when(jnp.logical_and(g < nb_ref[0], fb == NF))
    def _():
        out_ref[...] = lax.dot_general(
            hi_ref[...], wd_ref[...], dn,
            preferred_element_type=jnp.float32)


def _grouped_mlp(xs_pad, w0, w1, w2, s0, s1, s2, ebk, nb):
    def xs_map(g, fb, ebk_ref, nb_ref):
        return (jnp.minimum(g, nb_ref[0] - 1), 0)

    def w01_map(g, fb, ebk_ref, nb_ref):
        return (ebk_ref[g], jnp.minimum(fb, NF - 1), 0)

    def w2_map(g, fb, ebk_ref, nb_ref):
        return (ebk_ref[g], 0, jnp.minimum(fb, NF - 1))

    def s_map(g, fb, ebk_ref, nb_ref):
        return (ebk_ref[g], jnp.minimum(fb, NF - 1), 0, 0)

    def out_map(g, fb, ebk_ref, nb_ref):
        return (jnp.minimum(g, nb_ref[0] - 1), 0)

    grid_spec = pltpu.PrefetchScalarGridSpec(
        num_scalar_prefetch=2,
        grid=(G, NF + 1),
        in_specs=[
            pl.BlockSpec((BM, D), xs_map),
            pl.BlockSpec((1, BS, D), w01_map),
            pl.BlockSpec((1, BS, D), w01_map),
            pl.BlockSpec((1, D, BS), w2_map),
            pl.BlockSpec((1, 1, 1, D), s_map),
            pl.BlockSpec((1, 1, 1, D), s_map),
            pl.BlockSpec((1, 1, 1, D // BS), s_map),
        ],
        out_specs=pl.BlockSpec((BM, D), out_map),
        scratch_shapes=[
            pltpu.VMEM((BM, D), jnp.bfloat16),
            pltpu.VMEM((2, 2 * BS, D), jnp.bfloat16),
            pltpu.VMEM((BM, F), jnp.bfloat16),
            pltpu.VMEM((D, F), jnp.bfloat16),
        ],
    )
    s0r = jnp.repeat(s0, BS, axis=2).reshape(E, NF, 1, D)
    s1r = jnp.repeat(s1, BS, axis=2).reshape(E, NF, 1, D)
    s2r = s2.transpose(0, 2, 1).reshape(E, NF, 1, D // BS)
    return pl.pallas_call(
        _mlp_body,
        grid_spec=grid_spec,
        out_shape=jax.ShapeDtypeStruct((N, D), jnp.float32),
    )(ebk, nb, xs_pad, w0, w1, w2, s0r, s1r, s2r)


def kernel(x, selected_experts, w0, w1, w2, s0, s1, s2):
    se = selected_experts.reshape(P).astype(jnp.int32)

    order = jnp.argsort(se)
    counts = jnp.bincount(se, length=E)
    cstart = jnp.concatenate(
        [jnp.zeros((1,), jnp.int32), jnp.cumsum(counts)[:-1].astype(jnp.int32)])
    nblk = -(-counts // BM)
    blk_start = jnp.concatenate(
        [jnp.zeros((1,), jnp.int32), jnp.cumsum(nblk)[:-1].astype(jnp.int32)])
    nblocks = jnp.sum(nblk).astype(jnp.int32)
    pstart = blk_start * BM

    e_sorted = se[order]
    i = jnp.arange(P, dtype=jnp.int32)
    ppos = pstart[e_sorted] + (i - cstart[e_sorted])
    pad_fill = jnp.arange(N, dtype=jnp.int32) % T
    tok_map = pad_fill.at[ppos].set((order // K).astype(jnp.int32))
    pos_out = jnp.zeros((P,), jnp.int32).at[order].set(ppos)

    garr = jnp.arange(G, dtype=jnp.int32)
    raw_e = (jnp.searchsorted(blk_start, garr, side="right") - 1).astype(jnp.int32)
    last_e = raw_e[jnp.maximum(nblocks - 1, 0)]
    ebk = jnp.where(garr < nblocks, raw_e, last_e).astype(jnp.int32)
    nb = nblocks.reshape(1)

    xs_pad = _sc_row_gather(x, tok_map, N, 56, D, jnp.float32)
    ys_pad = _grouped_mlp(xs_pad, w0, w1, w2, s0, s1, s2, ebk, nb)
    out = _sc_row_gather(ys_pad, pos_out, P, 64, D, jnp.float32)
    return out.reshape(T, K, D)


# speedup vs baseline: 1.0655x; 1.0655x over previous
import functools

import jax
import jax.numpy as jnp
from jax import lax
from jax.experimental import pallas as pl
from jax.experimental.pallas import tpu as pltpu
from jax.experimental.pallas import tpu_sc as plsc

E = 8
K = 2
T = 2048
D = 1024
F = 1408
BS = 128

P = T * K
BM = 640
G = -(-P // BM) + E - 1
N = G * BM
NF = F // BS


def _sc_row_gather(src, idx, n_rows, chunk, width, dtype):
    info = plsc.get_sparse_core_info()
    nw = info.num_cores * info.num_subcores
    b_per_w = n_rows // nw
    n_chunks = b_per_w // chunk
    mesh = plsc.VectorSubcoreMesh(core_axis_name="c", subcore_axis_name="s")

    @functools.partial(
        pl.kernel,
        mesh=mesh,
        out_type=jax.ShapeDtypeStruct((n_rows, width), dtype),
        scratch_types=[
            pltpu.VMEM((b_per_w,), jnp.int32),
            pltpu.VMEM((chunk, width), dtype),
            pltpu.SemaphoreType.DMA,
        ],
    )
    def gather_k(src_hbm, idx_hbm, out_hbm, idx_v, buf, sem):
        wid = lax.axis_index("s") * info.num_cores + lax.axis_index("c")
        base = wid * b_per_w
        pltpu.sync_copy(idx_hbm.at[pl.ds(base, b_per_w)], idx_v)

        def body(c, _):
            off = c * chunk
            pltpu.async_copy(
                src_hbm.at[idx_v.at[pl.ds(off, chunk)]], buf, sem
            ).wait()
            pltpu.sync_copy(buf, out_hbm.at[pl.ds(base + off, chunk)])
            return 0

        lax.fori_loop(0, n_chunks, body, 0)

    return gather_k(src, idx)


def _mlp_body(ebk_ref, nb_ref, xs_ref, w0_ref, w1_ref, w2_ref,
              s0_ref, s1_ref, s2_ref, out_ref, xsb_ref, wgu_ref,
              hi_ref, wd_ref):
    g = pl.program_id(0)
    fb = pl.program_id(1)

    @pl.when(jnp.logical_and(g < nb_ref[0], fb == 0))
    def _():
        xsb_ref[...] = xs_ref[...].astype(jnp.bfloat16)

    dn = (((1,), (1,)), ((), ()))
    bf = jnp.bfloat16

    @pl.when(g < nb_ref[0])
    def _():
        xs = xsb_ref[...]
        s0_full = s0_ref[0, 0]
        s1_full = s1_ref[0, 0]
        expand = (lax.broadcasted_iota(jnp.int32, (D, D // BS), 0) // BS
                  == lax.broadcasted_iota(jnp.int32, (D, D // BS), 1)
                  ).astype(jnp.float32)
        s2_full = lax.dot_general(expand, s2_ref[0, 0], dn,
                                  preferred_element_type=jnp.float32)
        wgu_ref[0:BS, :] = (w0_ref[0] * s0_full).astype(bf)
        wgu_ref[BS:2 * BS, :] = (w1_ref[0] * s1_full).astype(bf)
        gu = lax.dot_general(xs, wgu_ref[...], dn,
                             preferred_element_type=jnp.float32)
        gi = gu[:, :BS]
        ui = gu[:, BS:]
        hi = (gi * jax.nn.sigmoid(gi) * ui).astype(bf)
        off = pl.multiple_of(fb * BS, BS)
        hi_ref[:, pl.ds(off, BS)] = hi
        wd_ref[:, pl.ds(off, BS)] = (w2_ref[0] * s2_full).astype(bf)

    @pl.when(jnp.logical_and(g < nb_ref[0], fb == NF - 1))
    def _():
        out_ref[...] = lax.dot_general(
            hi_ref[...], wd_ref[...], dn,
            preferred_element_type=jnp.float32)


def _grouped_mlp(xs_pad, w0, w1, w2, s0, s1, s2, ebk, nb):
    def xs_map(g, fb, ebk_ref, nb_ref):
        return (jnp.minimum(g, nb_ref[0] - 1), 0)

    def w01_map(g, fb, ebk_ref, nb_ref):
        return (ebk_ref[g], jnp.minimum(fb, NF - 1), 0)

    def w2_map(g, fb, ebk_ref, nb_ref):
        return (ebk_ref[g], 0, jnp.minimum(fb, NF - 1))

    def s_map(g, fb, ebk_ref, nb_ref):
        return (ebk_ref[g], jnp.minimum(fb, NF - 1), 0, 0)

    def out_map(g, fb, ebk_ref, nb_ref):
        return (jnp.minimum(g, nb_ref[0] - 1), 0)

    grid_spec = pltpu.PrefetchScalarGridSpec(
        num_scalar_prefetch=2,
        grid=(G, NF),
        in_specs=[
            pl.BlockSpec((BM, D), xs_map),
            pl.BlockSpec((1, BS, D), w01_map),
            pl.BlockSpec((1, BS, D), w01_map),
            pl.BlockSpec((1, D, BS), w2_map),
            pl.BlockSpec((1, 1, 1, D), s_map),
            pl.BlockSpec((1, 1, 1, D), s_map),
            pl.BlockSpec((1, 1, 1, D // BS), s_map),
        ],
        out_specs=pl.BlockSpec((BM, D), out_map),
        scratch_shapes=[
            pltpu.VMEM((BM, D), jnp.bfloat16),
            pltpu.VMEM((2 * BS, D), jnp.bfloat16),
            pltpu.VMEM((BM, F), jnp.bfloat16),
            pltpu.VMEM((D, F), jnp.bfloat16),
        ],
    )
    s0r = jnp.repeat(s0, BS, axis=2).reshape(E, NF, 1, D)
    s1r = jnp.repeat(s1, BS, axis=2).reshape(E, NF, 1, D)
    s2r = s2.transpose(0, 2, 1).reshape(E, NF, 1, D // BS)
    return pl.pallas_call(
        _mlp_body,
        grid_spec=grid_spec,
        out_shape=jax.ShapeDtypeStruct((N, D), jnp.float32),
    )(ebk, nb, xs_pad, w0, w1, w2, s0r, s1r, s2r)


def kernel(x, selected_experts, w0, w1, w2, s0, s1, s2):
    se = selected_experts.reshape(P).astype(jnp.int32)

    order = jnp.argsort(se)
    counts = jnp.bincount(se, length=E)
    cstart = jnp.concatenate(
        [jnp.zeros((1,), jnp.int32), jnp.cumsum(counts)[:-1].astype(jnp.int32)])
    nblk = -(-counts // BM)
    blk_start = jnp.concatenate(
        [jnp.zeros((1,), jnp.int32), jnp.cumsum(nblk)[:-1].astype(jnp.int32)])
    nblocks = jnp.sum(nblk).astype(jnp.int32)
    pstart = blk_start * BM

    e_sorted = se[order]
    i = jnp.arange(P, dtype=jnp.int32)
    ppos = pstart[e_sorted] + (i - cstart[e_sorted])
    pad_fill = jnp.arange(N, dtype=jnp.int32) % T
    tok_map = pad_fill.at[ppos].set((order // K).astype(jnp.int32))
    pos_out = jnp.zeros((P,), jnp.int32).at[order].set(ppos)

    garr = jnp.arange(G, dtype=jnp.int32)
    raw_e = (jnp.searchsorted(blk_start, garr, side="right") - 1).astype(jnp.int32)
    last_e = raw_e[jnp.maximum(nblocks - 1, 0)]
    ebk = jnp.where(garr < nblocks, raw_e, last_e).astype(jnp.int32)
    nb = nblocks.reshape(1)

    xs_pad = _sc_row_gather(x, tok_map, N, 56, D, jnp.float32)
    ys_pad = _grouped_mlp(xs_pad, w0, w1, w2, s0, s1, s2, ebk, nb)
    out = _sc_row_gather(ys_pad, pos_out, P, 64, D, jnp.float32)
    return out.reshape(T, K, D)
